# baseline (speedup 1.0000x reference)
import functools

import jax
import jax.numpy as jnp
from jax import lax
from jax.experimental import pallas as pl
from jax.experimental.pallas import tpu as pltpu
from jax.experimental.pallas import tpu_sc as plsc

N_FACTORS = 32
BATCH = 16384
LANES = 16
CHUNK = 128


def _build_kernel():
    info = plsc.get_sparse_core_info()
    nc, ns = info.num_cores, info.num_subcores
    nw = nc * ns
    b_per_w = BATCH // nw
    n_chunks = b_per_w // CHUNK
    n_groups = b_per_w // LANES

    mesh = plsc.VectorSubcoreMesh(core_axis_name="c", subcore_axis_name="s")

    @functools.partial(
        pl.kernel,
        mesh=mesh,
        compiler_params=pltpu.CompilerParams(
            needs_layout_passes=False, use_tc_tiling_on_sc=False),
        out_type=jax.ShapeDtypeStruct((BATCH,), jnp.float32),
        scratch_types=[
            pltpu.VMEM((n_chunks, CHUNK), jnp.int32),
            pltpu.VMEM((n_chunks, CHUNK), jnp.int32),
            pltpu.VMEM((b_per_w, N_FACTORS), jnp.float32),
            pltpu.VMEM((b_per_w, N_FACTORS), jnp.float32),
            pltpu.VMEM((b_per_w,), jnp.float32),
            pltpu.VMEM((b_per_w,), jnp.float32),
            pltpu.VMEM((LANES,), jnp.float32),
            pltpu.VMEM((b_per_w,), jnp.float32),
            pltpu.SemaphoreType.DMA,
        ],
    )
    def mf_kernel(user_hbm, item_hbm, uf_hbm, if_hbm, ub_hbm, ib_hbm,
                  gm_hbm, out_hbm,
                  idx_u, idx_i, uf_rows, if_rows, ub_rows, ib_rows,
                  gm_v, out_v, sem):
        wid = lax.axis_index("s") * nc + lax.axis_index("c")
        base = wid * b_per_w

        pltpu.sync_copy(gm_hbm, gm_v)
        for j in range(n_chunks):
            pltpu.sync_copy(user_hbm.at[pl.ds(base + j * CHUNK, CHUNK)],
                            idx_u.at[j])
            pltpu.sync_copy(item_hbm.at[pl.ds(base + j * CHUNK, CHUNK)],
                            idx_i.at[j])

        copies = []
        for j in range(n_chunks):
            sl = pl.ds(j * CHUNK, CHUNK)
            copies.append(pltpu.async_copy(
                uf_hbm.at[idx_u.at[j]], uf_rows.at[sl], sem))
            copies.append(pltpu.async_copy(
                if_hbm.at[idx_i.at[j]], if_rows.at[sl], sem))
            copies.append(pltpu.async_copy(
                ub_hbm.at[idx_u.at[j]], ub_rows.at[sl], sem))
            copies.append(pltpu.async_copy(
                ib_hbm.at[idx_i.at[j]], ib_rows.at[sl], sem))
        for c in copies:
            c.wait()

        gm = gm_v[...]
        lane = lax.iota(jnp.int32, LANES)

        def body(g, carry):
            base_r = g * LANES
            bu = ub_rows[pl.ds(base_r, LANES)]
            bi = ib_rows[pl.ds(base_r, LANES)]
            acc = gm + bu + bi
            for k in range(LANES):
                r = base_r + k
                s = (uf_rows[r, pl.ds(0, LANES)] * if_rows[r, pl.ds(0, LANES)]
                     + uf_rows[r, pl.ds(LANES, LANES)]
                     * if_rows[r, pl.ds(LANES, LANES)])
                dot_k = lax.reduce_sum_p.bind(s, axes=(0,))
                acc = jnp.where(lane == k, acc + dot_k, acc)
            out_v[pl.ds(base_r, LANES)] = acc
            return carry

        lax.fori_loop(0, n_groups, body, 0)
        pltpu.sync_copy(out_v, out_hbm.at[pl.ds(base, b_per_w)])

    return mf_kernel


def kernel(user, item, user_factors, item_factors, user_bias, item_bias,
           global_mean):
    gm16 = jnp.broadcast_to(
        jnp.asarray(global_mean, jnp.float32).reshape(()), (LANES,))
    mf = _build_kernel()
    return mf(user.astype(jnp.int32), item.astype(jnp.int32),
              user_factors, item_factors,
              user_bias.reshape(-1), item_bias.reshape(-1), gm16)
